# baseline (speedup 1.0000x reference)
import functools

import jax
import jax.numpy as jnp
from jax import lax
from jax.experimental import pallas as pl
from jax.experimental.pallas import tpu as pltpu
from jax.experimental.pallas import tpu_sc as plsc

NC = 2
NS = 16
NW = NC * NS
CHUNK = 128
NBUF = 8
RBUF = 2

_mesh = lambda: plsc.VectorSubcoreMesh(
    core_axis_name="c", subcore_axis_name="s",
    num_cores=NC, num_subcores=NS,
)


@functools.partial(jax.jit, static_argnames=("n_tiles", "lin_rows"))
def _sc_reformat(table_t, tail, *, n_tiles, lin_rows):
    tr = tail.shape[0]
    per_w = n_tiles // NW
    extra = n_tiles - per_w * NW

    @functools.partial(
        pl.kernel,
        out_type=jax.ShapeDtypeStruct((lin_rows, 128), jnp.float32),
        mesh=_mesh(),
        scratch_types=[
            pltpu.VMEM((RBUF, 32, 128), jnp.float32),
            pltpu.VMEM((RBUF, 32, 128), jnp.float32),
            pltpu.SemaphoreType.DMA((RBUF,)),
            pltpu.SemaphoreType.DMA((RBUF,)),
        ],
        compiler_params=pltpu.CompilerParams(use_tc_tiling_on_sc=True,
                                             needs_layout_passes=False),
    )
    def body(t2_hbm, tail_hbm, lin_hbm, in_v, out_v, isem, osem):
        wid = lax.axis_index("s") * NC + lax.axis_index("c")
        base = wid * per_w + jnp.minimum(wid, extra)
        u = per_w + (1 if extra else 0)
        u += (-u) % RBUF

        lane = lax.iota(jnp.int32, 16)

        def ti(i):
            return jnp.minimum(base + i, n_tiles - 1)

        def start_in(j, b):
            for e8 in range(4):
                pltpu.async_copy(
                    t2_hbm.at[pl.ds(e8 * 8, 8), pl.ds(j * 128, 128)],
                    in_v.at[b, pl.ds(e8 * 8, 8)], isem.at[b])

        def wait_in(j, b):
            for e8 in range(4):
                pltpu.make_async_copy(
                    t2_hbm.at[pl.ds(e8 * 8, 8), pl.ds(j * 128, 128)],
                    in_v.at[b, pl.ds(e8 * 8, 8)], isem.at[b]).wait()

        def start_out(j, b):
            pltpu.async_copy(out_v.at[b], lin_hbm.at[pl.ds(j * 32, 32)],
                             osem.at[b])

        def wait_out(j, b):
            pltpu.make_async_copy(out_v.at[b],
                                  lin_hbm.at[pl.ds(j * 32, 32)],
                                  osem.at[b]).wait()

        e_lo = lane
        e_hi = lane + 16

        def transpose(b):
            for r in range(32):
                for c0 in range(0, 128, 16):
                    e_pat = e_lo if (c0 % 32) == 0 else e_hi
                    cc = jnp.full((16,), 4 * r + c0 // 32, jnp.int32)
                    vec = plsc.load_gather(in_v.at[b], [e_pat, cc])
                    out_v[b, r, pl.ds(c0, 16)] = vec

        for b in range(RBUF):
            start_in(ti(b), b)
        for i in range(RBUF):
            j = ti(i)
            wait_in(j, i)
            transpose(i)
            start_out(j, i)
            start_in(ti(i + RBUF), i)

        @pl.loop(RBUF, u - RBUF, step=RBUF)
        def _(i0):
            for b in range(RBUF):
                i = i0 + b
                j = ti(i)
                wait_in(j, b)
                wait_out(ti(i - RBUF), b)
                transpose(b)
                start_out(j, b)
                start_in(ti(i + RBUF), b)

        for k in range(RBUF):
            i = u - RBUF + k
            j = ti(i)
            wait_in(j, k)
            wait_out(ti(i - RBUF), k)
            transpose(k)
            start_out(j, k)
        for k in range(RBUF):
            wait_out(ti(u - RBUF + k), k)

        @pl.when((wid == 0) & (tr > 0))
        def _():
            pltpu.sync_copy(tail_hbm, lin_hbm.at[pl.ds(lin_rows - tr, tr)])

    return body(table_t, tail)


@functools.partial(jax.jit, static_argnames=("n_chunks", "d"))
def _sc_gather(idx3, lin_table, *, n_chunks, d):
    rows_per_w = n_chunks * CHUNK

    @functools.partial(
        pl.kernel,
        out_type=jax.ShapeDtypeStruct((NW * rows_per_w, d), jnp.float32),
        mesh=_mesh(),
        scratch_types=[
            pltpu.VMEM((n_chunks, CHUNK), jnp.int32),
            pltpu.VMEM((NBUF, CHUNK, d), jnp.float32),
            pltpu.SemaphoreType.DMA((NBUF,)),
            pltpu.SemaphoreType.DMA((NBUF,)),
        ],
        compiler_params=pltpu.CompilerParams(use_tc_tiling_on_sc=False),
    )
    def body(table_hbm, idx_hbm, out_hbm, idx_v, rows_v, gsem, osem):
        wid = lax.axis_index("s") * NC + lax.axis_index("c")
        base = wid * rows_per_w
        pltpu.sync_copy(idx_hbm.at[wid], idx_v)

        def start_gather(c, b):
            pltpu.async_copy(table_hbm.at[idx_v.at[c]], rows_v.at[b],
                             gsem.at[b])

        def wait_gather(c, b):
            pltpu.make_async_copy(table_hbm.at[idx_v.at[c]], rows_v.at[b],
                                  gsem.at[b]).wait()

        def start_out(c, b):
            pltpu.async_copy(rows_v.at[b],
                             out_hbm.at[pl.ds(base + c * CHUNK, CHUNK)],
                             osem.at[b])

        def wait_out(c, b):
            pltpu.make_async_copy(rows_v.at[b],
                                  out_hbm.at[pl.ds(base + c * CHUNK, CHUNK)],
                                  osem.at[b]).wait()

        for b in range(NBUF):
            start_gather(b, b)

        @pl.loop(0, n_chunks - NBUF, step=NBUF)
        def _(c0):
            for b in range(NBUF):
                c = c0 + b
                wait_gather(c, b)
                start_out(c, b)
                wait_out(c, b)
                start_gather(c + NBUF, b)

        for b in range(NBUF):
            c = n_chunks - NBUF + b
            wait_gather(c, b)
            start_out(c, b)
            wait_out(c, b)

    return body(lin_table, idx3)


def kernel(x, table):
    b, h = x.shape
    v, d = table.shape
    r = b * h
    idx = x.reshape(-1).astype(jnp.int32)

    n_tiles = v // 128
    v_main = n_tiles * 128
    tail2 = table[v_main:, :].reshape(-1, 128) if v_main < v else \
        jnp.zeros((0, 128), jnp.float32)
    lin_rows = (v * d) // 128
    lin = _sc_reformat(jnp.swapaxes(table, 0, 1), tail2,
                       n_tiles=n_tiles, lin_rows=lin_rows)
    lin2 = lin.reshape(v, d)

    grain = NW * CHUNK * NBUF
    r_pad = ((r + grain - 1) // grain) * grain
    if r_pad != r:
        idx = jnp.concatenate([idx, jnp.zeros((r_pad - r,), jnp.int32)])
    n_chunks = r_pad // (NW * CHUNK)
    idx3 = idx.reshape(NW, n_chunks, CHUNK)

    out = _sc_gather(idx3, lin2, n_chunks=n_chunks, d=d)
    return out[:r].reshape(b, h, d)


# speedup vs baseline: 1.5050x; 1.5050x over previous
import functools

import jax
import jax.numpy as jnp
from jax import lax
from jax.experimental import pallas as pl
from jax.experimental.pallas import tpu as pltpu
from jax.experimental.pallas import tpu_sc as plsc

NC = 2
NS = 16
NW = NC * NS
CHUNK = 128
NBUF = 8
RBUF = 2

_mesh = lambda: plsc.VectorSubcoreMesh(
    core_axis_name="c", subcore_axis_name="s",
    num_cores=NC, num_subcores=NS,
)


@functools.partial(jax.jit, static_argnames=("n_tiles", "lin_rows"))
def _sc_reformat(table_t, tail, *, n_tiles, lin_rows):
    tr = tail.shape[0]
    per_w = n_tiles // NW
    extra = n_tiles - per_w * NW

    @functools.partial(
        pl.kernel,
        out_type=jax.ShapeDtypeStruct((lin_rows, 128), jnp.float32),
        mesh=_mesh(),
        scratch_types=[
            pltpu.VMEM((RBUF, 32, 128), jnp.float32),
            pltpu.VMEM((RBUF, 32, 128), jnp.float32),
            pltpu.SemaphoreType.DMA((RBUF,)),
            pltpu.SemaphoreType.DMA((RBUF,)),
        ],
        compiler_params=pltpu.CompilerParams(use_tc_tiling_on_sc=True,
                                             needs_layout_passes=False),
    )
    def body(t2_hbm, tail_hbm, lin_hbm, in_v, out_v, isem, osem):
        wid = lax.axis_index("s") * NC + lax.axis_index("c")
        base = wid * per_w + jnp.minimum(wid, extra)
        u = per_w + (1 if extra else 0)
        u += (-u) % RBUF

        lane = lax.iota(jnp.int32, 16)

        def ti(i):
            return jnp.minimum(base + i, n_tiles - 1)

        def start_in(j, b):
            for e8 in range(4):
                pltpu.async_copy(
                    t2_hbm.at[pl.ds(e8 * 8, 8), pl.ds(j * 128, 128)],
                    in_v.at[b, pl.ds(e8 * 8, 8)], isem.at[b])

        def wait_in(j, b):
            for e8 in range(4):
                pltpu.make_async_copy(
                    t2_hbm.at[pl.ds(e8 * 8, 8), pl.ds(j * 128, 128)],
                    in_v.at[b, pl.ds(e8 * 8, 8)], isem.at[b]).wait()

        def start_out(j, b):
            pltpu.async_copy(out_v.at[b], lin_hbm.at[pl.ds(j * 32, 32)],
                             osem.at[b])

        def wait_out(j, b):
            pltpu.make_async_copy(out_v.at[b],
                                  lin_hbm.at[pl.ds(j * 32, 32)],
                                  osem.at[b]).wait()

        e_lo = lane
        e_hi = lane + 16

        def transpose(b):
            @plsc.parallel_loop(0, 32, step=1, unroll=4)
            def _(r):
                for c0 in range(0, 128, 16):
                    e_pat = e_lo if (c0 % 32) == 0 else e_hi
                    cc = jnp.full((16,), 1, jnp.int32) * (4 * r + c0 // 32)
                    vec = plsc.load_gather(in_v.at[b], [e_pat, cc])
                    out_v[b, r, pl.ds(c0, 16)] = vec

        for b in range(RBUF):
            start_in(ti(b), b)
        for i in range(RBUF):
            j = ti(i)
            wait_in(j, i)
            transpose(i)
            start_out(j, i)
            start_in(ti(i + RBUF), i)

        @pl.loop(RBUF, u - RBUF, step=RBUF)
        def _(i0):
            for b in range(RBUF):
                i = i0 + b
                j = ti(i)
                wait_in(j, b)
                wait_out(ti(i - RBUF), b)
                transpose(b)
                start_out(j, b)
                start_in(ti(i + RBUF), b)

        for k in range(RBUF):
            i = u - RBUF + k
            j = ti(i)
            wait_in(j, k)
            wait_out(ti(i - RBUF), k)
            transpose(k)
            start_out(j, k)
        for k in range(RBUF):
            wait_out(ti(u - RBUF + k), k)

        @pl.when((wid == 0) & (tr > 0))
        def _():
            pltpu.sync_copy(tail_hbm, lin_hbm.at[pl.ds(lin_rows - tr, tr)])

    return body(table_t, tail)


@functools.partial(jax.jit, static_argnames=("n_chunks", "d"))
def _sc_gather(idx3, lin_table, *, n_chunks, d):
    rows_per_w = n_chunks * CHUNK

    @functools.partial(
        pl.kernel,
        out_type=jax.ShapeDtypeStruct((NW * rows_per_w, d), jnp.float32),
        mesh=_mesh(),
        scratch_types=[
            pltpu.VMEM((n_chunks, CHUNK), jnp.int32),
            pltpu.VMEM((NBUF, CHUNK, d), jnp.float32),
            pltpu.SemaphoreType.DMA((NBUF,)),
            pltpu.SemaphoreType.DMA((NBUF,)),
        ],
        compiler_params=pltpu.CompilerParams(use_tc_tiling_on_sc=False),
    )
    def body(table_hbm, idx_hbm, out_hbm, idx_v, rows_v, gsem, osem):
        wid = lax.axis_index("s") * NC + lax.axis_index("c")
        base = wid * rows_per_w
        pltpu.sync_copy(idx_hbm.at[wid], idx_v)

        def start_gather(c, b):
            pltpu.async_copy(table_hbm.at[idx_v.at[c]], rows_v.at[b],
                             gsem.at[b])

        def wait_gather(c, b):
            pltpu.make_async_copy(table_hbm.at[idx_v.at[c]], rows_v.at[b],
                                  gsem.at[b]).wait()

        def start_out(c, b):
            pltpu.async_copy(rows_v.at[b],
                             out_hbm.at[pl.ds(base + c * CHUNK, CHUNK)],
                             osem.at[b])

        def wait_out(c, b):
            pltpu.make_async_copy(rows_v.at[b],
                                  out_hbm.at[pl.ds(base + c * CHUNK, CHUNK)],
                                  osem.at[b]).wait()

        for b in range(NBUF):
            start_gather(b, b)

        @pl.loop(0, n_chunks - NBUF, step=NBUF)
        def _(c0):
            for b in range(NBUF):
                c = c0 + b
                wait_gather(c, b)
                start_out(c, b)
                wait_out(c, b)
                start_gather(c + NBUF, b)

        for b in range(NBUF):
            c = n_chunks - NBUF + b
            wait_gather(c, b)
            start_out(c, b)
            wait_out(c, b)

    return body(lin_table, idx3)


def kernel(x, table):
    b, h = x.shape
    v, d = table.shape
    r = b * h
    idx = x.reshape(-1).astype(jnp.int32)

    n_tiles = v // 128
    v_main = n_tiles * 128
    tail2 = table[v_main:, :].reshape(-1, 128) if v_main < v else \
        jnp.zeros((0, 128), jnp.float32)
    lin_rows = (v * d) // 128
    lin = _sc_reformat(jnp.swapaxes(table, 0, 1), tail2,
                       n_tiles=n_tiles, lin_rows=lin_rows)
    lin2 = lin.reshape(v, d)

    grain = NW * CHUNK * NBUF
    r_pad = ((r + grain - 1) // grain) * grain
    if r_pad != r:
        idx = jnp.concatenate([idx, jnp.zeros((r_pad - r,), jnp.int32)])
    n_chunks = r_pad // (NW * CHUNK)
    idx3 = idx.reshape(NW, n_chunks, CHUNK)

    out = _sc_gather(idx3, lin2, n_chunks=n_chunks, d=d)
    return out[:r].reshape(b, h, d)
